# baseline (speedup 1.0000x reference)
import jax
import jax.numpy as jnp
from jax import lax
from jax.experimental import pallas as pl
from jax.experimental.pallas import tpu as pltpu
from jax.experimental.pallas import tpu_sc as plsc

_NO_CAT = 100000
_EMBED_DIM = 64
_BATCH = 4096
_FIELDS = 26

_B = _BATCH * _FIELDS
_NC = 2
_NS = 16
_NW = _NC * _NS
_BL = _BATCH // _NW
_B_PER_W = _B // _NW
_NPASS = _FIELDS // 2
_PROWS = 2 * _BL


def _kern(table_hbm, idx_hbm, out_hbm, idx_v, sl_v, g_v, b_v, *sems):
    gsems, wsems = sems[:2], sems[2:]
    wid = lax.axis_index("s") * _NC + lax.axis_index("c")
    base = wid * _B_PER_W
    pltpu.sync_copy(idx_hbm.at[pl.ds(base, _B_PER_W)], idx_v)

    lane = lax.iota(jnp.int32, 16)

    def build_sublist(p, s):
        for fi in range(2):
            f = 2 * p + fi
            for blg in range(8):
                addr = lane * _FIELDS + (blg * 16 * _FIELDS + f)
                vals = plsc.load_gather(idx_v, [addr])
                sl_v[s, pl.ds(fi * _BL + blg * 16, 16)] = vals

    def g_desc(s):
        return pltpu.make_async_copy(
            table_hbm.at[sl_v.at[s]], g_v.at[s], gsems[s]
        )

    def transpose(s):
        @pl.loop(0, _EMBED_DIM)
        def _(dd):
            dt = lax.shift_right_logical(dd, 2)
            dt = lax.shift_right_logical(dt, 1)
            dr = lax.bitwise_and(dd, 7)
            d_idx = lane * 0 + dd
            for fi in range(2):
                for blg in range(8):
                    j = lane + (fi * _BL + blg * 16)
                    vals = plsc.load_gather(g_v.at[s], [j, d_idx])
                    b_v[s, fi, dt, dr, pl.ds(blg * 16, 16)] = vals

    def w_descs(p, s):
        return [
            pltpu.make_async_copy(
                b_v.at[s, fi, dt], out_hbm.at[2 * p + fi, dt, wid], wsems[s]
            )
            for fi in range(2)
            for dt in range(8)
        ]

    build_sublist(0, 0)
    g_desc(0).start()
    for p in range(_NPASS):
        s = p % 2
        g_desc(s).wait()
        if p + 1 < _NPASS:
            build_sublist(p + 1, 1 - s)
            g_desc(1 - s).start()
        if p >= 2:
            for d in w_descs(p - 2, s):
                d.wait()
        transpose(s)
        for d in w_descs(p, s):
            d.start()
    for p in (_NPASS - 2, _NPASS - 1):
        for d in w_descs(p, p % 2):
            d.wait()


@jax.jit
def _embedding_lookup(idx_flat, table):
    mesh = plsc.VectorSubcoreMesh(core_axis_name="c", subcore_axis_name="s")
    run = pl.kernel(
        _kern,
        out_type=jax.ShapeDtypeStruct(
            (_FIELDS, 8, _NW, 8, 128), jnp.float32
        ),
        mesh=mesh,
        scratch_types=[
            pltpu.VMEM((_B_PER_W,), jnp.int32),
            pltpu.VMEM((2, _PROWS), jnp.int32),
            pltpu.VMEM((2, _PROWS, _EMBED_DIM), jnp.float32),
            pltpu.VMEM((2, 2, 8, 8, 128), jnp.float32),
        ] + [pltpu.SemaphoreType.DMA] * 4,
        compiler_params=pltpu.CompilerParams(
            use_tc_tiling_on_sc=False, needs_layout_passes=False
        ),
    )
    return run(table, idx_flat)


def kernel(x, table):
    idx_flat = x.reshape(_B).astype(jnp.int32)
    p = _embedding_lookup(idx_flat, table)
    return p.transpose(2, 4, 0, 1, 3).reshape(_BATCH, _FIELDS, _EMBED_DIM)


# speedup vs baseline: 1.4212x; 1.4212x over previous
import jax
import jax.numpy as jnp
from jax import lax
from jax.experimental import pallas as pl
from jax.experimental.pallas import tpu as pltpu
from jax.experimental.pallas import tpu_sc as plsc

_NO_CAT = 100000
_EMBED_DIM = 64
_BATCH = 4096
_FIELDS = 26

_B = _BATCH * _FIELDS
_NC = 2
_NS = 16
_NW = _NC * _NS
_BL = _BATCH // _NW
_B_PER_W = _B // _NW
_NPASS = _FIELDS // 2
_PROWS = 2 * _BL


def _kern(table_hbm, idx_hbm, out_hbm, idx_v, sl_v, g_v, b_v, *sems):
    gsems, wsems = sems[:2], sems[2:]
    wid = lax.axis_index("s") * _NC + lax.axis_index("c")
    base = wid * _B_PER_W
    pltpu.sync_copy(idx_hbm.at[pl.ds(base, _B_PER_W)], idx_v)

    lane = lax.iota(jnp.int32, 16)

    def build_sublist(p, s):
        for fi in range(2):
            f = 2 * p + fi
            for blg in range(8):
                addr = lane * _FIELDS + (blg * 16 * _FIELDS + f)
                vals = plsc.load_gather(idx_v, [addr])
                sl_v[s, pl.ds(fi * _BL + blg * 16, 16)] = vals

    def g_desc(s):
        return pltpu.make_async_copy(
            table_hbm.at[sl_v.at[s]], g_v.at[s], gsems[s]
        )

    j_vecs = [
        [lane + (fi * _BL + blg * 16) for blg in range(8)] for fi in range(2)
    ]

    def transpose(s):
        @plsc.parallel_loop(0, _EMBED_DIM, unroll=2)
        def _(dd):
            d_idx = lane * 0 + dd
            for fi in range(2):
                for blg in range(8):
                    vals = plsc.load_gather(g_v.at[s], [j_vecs[fi][blg], d_idx])
                    b_v[s, fi, dd, pl.ds(blg * 16, 16)] = vals

    def w_descs(p, s):
        return [
            pltpu.make_async_copy(
                b_v.at[s, fi, pl.ds(dt * 8, 8)],
                out_hbm.at[2 * p + fi, dt, wid],
                wsems[s],
            )
            for fi in range(2)
            for dt in range(8)
        ]

    build_sublist(0, 0)
    g_desc(0).start()
    for p in range(_NPASS):
        s = p % 2
        g_desc(s).wait()
        if p + 1 < _NPASS:
            build_sublist(p + 1, 1 - s)
            g_desc(1 - s).start()
        if p >= 2:
            for d in w_descs(p - 2, s):
                d.wait()
        transpose(s)
        for d in w_descs(p, s):
            d.start()
    for p in (_NPASS - 2, _NPASS - 1):
        for d in w_descs(p, p % 2):
            d.wait()


@jax.jit
def _embedding_lookup(idx_flat, table):
    mesh = plsc.VectorSubcoreMesh(core_axis_name="c", subcore_axis_name="s")
    run = pl.kernel(
        _kern,
        out_type=jax.ShapeDtypeStruct(
            (_FIELDS, 8, _NW, 8, 128), jnp.float32
        ),
        mesh=mesh,
        scratch_types=[
            pltpu.VMEM((_B_PER_W,), jnp.int32),
            pltpu.VMEM((2, _PROWS), jnp.int32),
            pltpu.VMEM((2, _PROWS, _EMBED_DIM), jnp.float32),
            pltpu.VMEM((2, 2, _EMBED_DIM, 128), jnp.float32),
        ] + [pltpu.SemaphoreType.DMA] * 4,
        compiler_params=pltpu.CompilerParams(
            use_tc_tiling_on_sc=False, needs_layout_passes=False
        ),
    )
    return run(table, idx_flat)


def kernel(x, table):
    idx_flat = x.reshape(_B).astype(jnp.int32)
    p = _embedding_lookup(idx_flat, table)
    return p.transpose(2, 4, 0, 1, 3).reshape(_BATCH, _FIELDS, _EMBED_DIM)
